# baseline (speedup 1.0000x reference)
import jax
import jax.numpy as jnp
from jax.experimental import pallas as pl
from jax.experimental.pallas import tpu as pltpu

_BLK = 5000


def _fill_body(x_ref, o_ref):
    x = x_ref[...]
    safe = jnp.ones_like(x, dtype=jnp.bool_)
    o_ref[...] = jnp.where(~safe, jnp.float32(-jnp.inf), x)


def kernel(logits, attention_mask):
    B, V = logits.shape
    flat = logits.reshape(-1, 256)
    R = flat.shape[0]
    out = pl.pallas_call(
        _fill_body,
        grid=(R // _BLK,),
        in_specs=[pl.BlockSpec((_BLK, 256), lambda i: (i, 0))],
        out_specs=pl.BlockSpec((_BLK, 256), lambda i: (i, 0)),
        out_shape=jax.ShapeDtypeStruct((R, 256), jnp.float32),
        compiler_params=pltpu.CompilerParams(
            dimension_semantics=("arbitrary",),
        ),
    )(flat)
    return out.reshape(B, V)


# speedup vs baseline: 7.9752x; 7.9752x over previous
import jax
import jax.numpy as jnp
from jax.experimental import pallas as pl
from jax.experimental.pallas import tpu as pltpu

_BV = 12800


def _fill_body(x_ref, o_ref):
    x = x_ref[...]
    safe = jnp.ones_like(x, dtype=jnp.bool_)
    o_ref[...] = jnp.where(~safe, jnp.float32(-jnp.inf), x)


def kernel(logits, attention_mask):
    B, V = logits.shape
    out = pl.pallas_call(
        _fill_body,
        grid=(pl.cdiv(V, _BV),),
        in_specs=[pl.BlockSpec((B, _BV), lambda i: (0, i))],
        out_specs=pl.BlockSpec((B, _BV), lambda i: (0, i)),
        out_shape=jax.ShapeDtypeStruct((B, V), jnp.float32),
        compiler_params=pltpu.CompilerParams(
            dimension_semantics=("arbitrary",),
        ),
    )(logits)
    return out


# speedup vs baseline: 8.0677x; 1.0116x over previous
import jax
import jax.numpy as jnp
from jax.experimental import pallas as pl
from jax.experimental.pallas import tpu as pltpu

_BR = 8


def _fill_body(x_ref, o_ref):
    x = x_ref[...]
    safe = jnp.ones_like(x, dtype=jnp.bool_)
    o_ref[...] = jnp.where(~safe, jnp.float32(-jnp.inf), x)


def kernel(logits, attention_mask):
    B, V = logits.shape
    out = pl.pallas_call(
        _fill_body,
        grid=(B // _BR,),
        in_specs=[pl.BlockSpec((_BR, V), lambda i: (i, 0))],
        out_specs=pl.BlockSpec((_BR, V), lambda i: (i, 0)),
        out_shape=jax.ShapeDtypeStruct((B, V), jnp.float32),
        compiler_params=pltpu.CompilerParams(
            dimension_semantics=("arbitrary",),
        ),
    )(logits)
    return out


# speedup vs baseline: 8.7058x; 1.0791x over previous
import jax
import jax.numpy as jnp
from jax.experimental import pallas as pl
from jax.experimental.pallas import tpu as pltpu

_BR = 16


def _fill_body(x_ref, o_ref):
    x = x_ref[...]
    safe = jnp.ones_like(x, dtype=jnp.bool_)
    o_ref[...] = jnp.where(~safe, jnp.float32(-jnp.inf), x)


def kernel(logits, attention_mask):
    B, V = logits.shape
    out = pl.pallas_call(
        _fill_body,
        grid=(B // _BR,),
        in_specs=[pl.BlockSpec((_BR, V), lambda i: (i, 0))],
        out_specs=pl.BlockSpec((_BR, V), lambda i: (i, 0)),
        out_shape=jax.ShapeDtypeStruct((B, V), jnp.float32),
        compiler_params=pltpu.CompilerParams(
            dimension_semantics=("arbitrary",),
        ),
    )(logits)
    return out


# speedup vs baseline: 9.1168x; 1.0472x over previous
import jax
import jax.numpy as jnp
from jax.experimental import pallas as pl
from jax.experimental.pallas import tpu as pltpu

_BR = 32


def _fill_body(x_ref, o_ref):
    x = x_ref[...]
    safe = jnp.ones_like(x, dtype=jnp.bool_)
    o_ref[...] = jnp.where(~safe, jnp.float32(-jnp.inf), x)


def kernel(logits, attention_mask):
    B, V = logits.shape
    out = pl.pallas_call(
        _fill_body,
        grid=(B // _BR,),
        in_specs=[pl.BlockSpec((_BR, V), lambda i: (i, 0))],
        out_specs=pl.BlockSpec((_BR, V), lambda i: (i, 0)),
        out_shape=jax.ShapeDtypeStruct((B, V), jnp.float32),
        compiler_params=pltpu.CompilerParams(
            dimension_semantics=("arbitrary",),
        ),
    )(logits)
    return out
